# Initial kernel scaffold; baseline (speedup 1.0000x reference)
import jax
import jax.numpy as jnp
from jax.experimental import pallas as pl


def kernel(x, upsample_indices):
    raise NotImplementedError("write your pallas kernel here")


# speedup vs baseline: 2.7859x; 2.7859x over previous
import functools

import jax
import jax.numpy as jnp
from jax import lax
from jax.experimental import pallas as pl
from jax.experimental.pallas import tpu as pltpu
from jax.experimental.pallas import tpu_sc as plsc

TARGET = 98304
NROWS = 65536
NUP = TARGET - NROWS
D = 128
L = 16

NC, NS = 2, 16
NW = NC * NS
UP_PER_W = NUP // NW
CP_PER_W = NROWS // NW
GB = 128
CB = 256

_MESH = plsc.VectorSubcoreMesh(
    core_axis_name="c", subcore_axis_name="s", num_cores=NC, num_subcores=NS
)


@functools.partial(
    pl.kernel,
    out_type=jax.ShapeDtypeStruct((TARGET, D), jnp.float32),
    mesh=_MESH,
    scratch_types=[
        pltpu.VMEM((UP_PER_W,), jnp.int32),
        pltpu.VMEM((UP_PER_W,), jnp.int32),
        pltpu.VMEM((GB, D), jnp.float32),
        pltpu.VMEM((GB, D), jnp.float32),
        pltpu.VMEM((GB, D), jnp.float32),
        pltpu.VMEM((CB, D), jnp.float32),
        pltpu.SemaphoreType.DMA,
        pltpu.SemaphoreType.DMA,
    ],
)
def _hex_unpool(x_hbm, idx0_hbm, idx1_hbm, out_hbm, i0v, i1v, r0, r1, ob, cb, s0, s1):
    wid = lax.axis_index("s") * NC + lax.axis_index("c")

    cbase = wid * CP_PER_W

    def copy_body(c, carry):
        row = cbase + c * CB
        pltpu.sync_copy(x_hbm.at[pl.ds(row, CB)], cb)
        pltpu.sync_copy(cb, out_hbm.at[pl.ds(row, CB)])
        return carry

    lax.fori_loop(0, CP_PER_W // CB, copy_body, 0)

    ubase = wid * UP_PER_W
    pltpu.sync_copy(idx0_hbm.at[pl.ds(ubase, UP_PER_W)], i0v)
    pltpu.sync_copy(idx1_hbm.at[pl.ds(ubase, UP_PER_W)], i1v)

    for j in range(UP_PER_W // GB):
        d0 = pltpu.async_copy(x_hbm.at[i0v.at[pl.ds(j * GB, GB)]], r0, s0)
        d1 = pltpu.async_copy(x_hbm.at[i1v.at[pl.ds(j * GB, GB)]], r1, s1)
        d0.wait()
        d1.wait()

        def avg_body(r, carry):
            for c in range(D // L):
                a = r0[r, pl.ds(c * L, L)]
                b = r1[r, pl.ds(c * L, L)]
                ob[r, pl.ds(c * L, L)] = (a + b) * 0.5
            return carry

        lax.fori_loop(0, GB, avg_body, 0)
        pltpu.sync_copy(ob, out_hbm.at[pl.ds(NROWS + ubase + j * GB, GB)])


def kernel(x, upsample_indices):
    idx0 = upsample_indices[:, 0]
    idx1 = upsample_indices[:, 1]
    return _hex_unpool(x, idx0, idx1)
